# baseline (speedup 1.0000x reference)
import functools

import jax
import jax.numpy as jnp
from jax import lax
from jax.experimental import pallas as pl
from jax.experimental.pallas import tpu as pltpu
from jax.experimental.pallas import tpu_sc as plsc

N = 4096
EMB = 128
K1 = 17
KN = K1 - 1
E = KN * N
ROWS = 512
NBLK = N // ROWS
EBLK = 16
F32 = jnp.float32
I32 = jnp.int32


def _lin_in_body(pos_ref, w_ref, b_ref, out_ref):
    out_ref[...] = (
        jnp.dot(pos_ref[...], w_ref[...], preferred_element_type=F32)
        + b_ref[...]
    )


def _lin_in(pos_p, w_p, b):
    return pl.pallas_call(
        _lin_in_body,
        out_shape=jax.ShapeDtypeStruct((N, EMB), F32),
    )(pos_p, w_p, b)


def _knn_body(h_ref, ht_ref, w1t_ref, w1b_ref, b1_ref, idx_ref, a_ref, b_ref):
    hb = h_ref[...]
    a_ref[...] = jnp.dot(hb, w1t_ref[...], preferred_element_type=F32)
    b_ref[...] = (
        jnp.dot(hb, w1b_ref[...], preferred_element_type=F32) + b1_ref[...]
    )
    ht = ht_ref[...]
    sq = jnp.sum(ht * ht, axis=0, keepdims=True)
    d = sq - 2.0 * jnp.dot(hb, ht, preferred_element_type=F32)
    iota = lax.broadcasted_iota(I32, (ROWS, N), 1)
    lane = lax.broadcasted_iota(I32, (ROWS, 128), 1)
    acc = jnp.zeros((ROWS, 128), I32)
    for j in range(K1):
        m = jnp.min(d, axis=1, keepdims=True)
        cand = jnp.where(d <= m, iota, 1 << 30)
        a = jnp.min(cand, axis=1, keepdims=True)
        acc = jnp.where(lane == j, a, acc)
        d = jnp.where(iota == a, 1.0e30, d)
    idx_ref[...] = acc


def _knn(h, ht, w1t, w1b, b1):
    return pl.pallas_call(
        _knn_body,
        grid=(NBLK,),
        compiler_params=pltpu.CompilerParams(
            dimension_semantics=("parallel",)),
        in_specs=[
            pl.BlockSpec((ROWS, EMB), lambda i: (i, 0)),
            pl.BlockSpec((EMB, N), lambda i: (0, 0)),
            pl.BlockSpec((EMB, EMB), lambda i: (0, 0)),
            pl.BlockSpec((EMB, EMB), lambda i: (0, 0)),
            pl.BlockSpec((1, EMB), lambda i: (0, 0)),
        ],
        out_specs=[
            pl.BlockSpec((ROWS, 128), lambda i: (i, 0)),
            pl.BlockSpec((ROWS, EMB), lambda i: (i, 0)),
            pl.BlockSpec((ROWS, EMB), lambda i: (i, 0)),
        ],
        out_shape=[
            jax.ShapeDtypeStruct((N, 128), I32),
            jax.ShapeDtypeStruct((N, EMB), F32),
            jax.ShapeDtypeStruct((N, EMB), F32),
        ],
    )(h, ht, w1t, w1b, b1)


def _stats1_body(xg_ref, bs_ref, st_ref):
    i = pl.program_id(0)

    @pl.when(i == 0)
    def _():
        st_ref[...] = jnp.zeros((8, EMB), F32)

    x = xg_ref[...] + bs_ref[...]
    st_ref[0:1, :] += jnp.sum(x, axis=0, keepdims=True)
    st_ref[1:2, :] += jnp.sum(x * x, axis=0, keepdims=True)


def _stats1(xg, bs):
    return pl.pallas_call(
        _stats1_body,
        grid=(EBLK,),
        in_specs=[
            pl.BlockSpec((N, EMB), lambda i: (i, 0)),
            pl.BlockSpec((N, EMB), lambda i: (0, 0)),
        ],
        out_specs=pl.BlockSpec((8, EMB), lambda i: (0, 0)),
        out_shape=jax.ShapeDtypeStruct((8, EMB), F32),
    )(xg, bs)


def _bn_coeffs(st_ref, g_ref, be_ref, count):
    m = st_ref[0:1, :] * (1.0 / count)
    v = st_ref[1:2, :] * (1.0 / count) - m * m
    sc = g_ref[...] * lax.rsqrt(v + 1e-5)
    sh = be_ref[...] - m * sc
    return sc, sh


def _p2_body(xg_ref, bs_ref, st_ref, g_ref, be_ref, w2_ref, b2_ref,
             y_ref, st2_ref):
    i = pl.program_id(0)

    @pl.when(i == 0)
    def _():
        st2_ref[...] = jnp.zeros((8, EMB), F32)

    sc, sh = _bn_coeffs(st_ref, g_ref, be_ref, float(E))
    x = xg_ref[...] + bs_ref[...]
    xh = jnp.maximum(x * sc + sh, 0.0)
    y = jnp.dot(xh, w2_ref[...], preferred_element_type=F32) + b2_ref[...]
    y_ref[...] = y
    st2_ref[0:1, :] += jnp.sum(y, axis=0, keepdims=True)
    st2_ref[1:2, :] += jnp.sum(y * y, axis=0, keepdims=True)


def _p2(xg, bs, st1, g1, be1, w2, b2):
    return pl.pallas_call(
        _p2_body,
        grid=(EBLK,),
        in_specs=[
            pl.BlockSpec((N, EMB), lambda i: (i, 0)),
            pl.BlockSpec((N, EMB), lambda i: (0, 0)),
            pl.BlockSpec((8, EMB), lambda i: (0, 0)),
            pl.BlockSpec((1, EMB), lambda i: (0, 0)),
            pl.BlockSpec((1, EMB), lambda i: (0, 0)),
            pl.BlockSpec((EMB, EMB), lambda i: (0, 0)),
            pl.BlockSpec((1, EMB), lambda i: (0, 0)),
        ],
        out_specs=[
            pl.BlockSpec((N, EMB), lambda i: (i, 0)),
            pl.BlockSpec((8, EMB), lambda i: (0, 0)),
        ],
        out_shape=[
            jax.ShapeDtypeStruct((E, EMB), F32),
            jax.ShapeDtypeStruct((8, EMB), F32),
        ],
    )(xg, bs, st1, g1, be1, w2, b2)


def _p3_body(y_ref, st_ref, g_ref, be_ref, msg_ref):
    sc, sh = _bn_coeffs(st_ref, g_ref, be_ref, float(E))
    msg_ref[...] = jnp.maximum(y_ref[...] * sc + sh, 0.0)


def _p3(y, st2, g2, be2):
    return pl.pallas_call(
        _p3_body,
        grid=(EBLK,),
        compiler_params=pltpu.CompilerParams(
            dimension_semantics=("parallel",)),
        in_specs=[
            pl.BlockSpec((N, EMB), lambda i: (i, 0)),
            pl.BlockSpec((8, EMB), lambda i: (0, 0)),
            pl.BlockSpec((1, EMB), lambda i: (0, 0)),
            pl.BlockSpec((1, EMB), lambda i: (0, 0)),
        ],
        out_specs=pl.BlockSpec((N, EMB), lambda i: (i, 0)),
        out_shape=jax.ShapeDtypeStruct((E, EMB), F32),
    )(y, st2, g2, be2)


def _update_body(h_ref, ag_ref, u1t_ref, u1b_ref, b1_ref, g1_ref, be1_ref,
                 u2_ref, b2_ref, g2_ref, be2_ref, out_ref):
    h = h_ref[...]
    aggr = ag_ref[0] + ag_ref[1]
    x = (
        jnp.dot(h, u1t_ref[...], preferred_element_type=F32)
        + jnp.dot(aggr, u1b_ref[...], preferred_element_type=F32)
        + b1_ref[...]
    )
    m = jnp.sum(x, axis=0, keepdims=True) * (1.0 / N)
    v = jnp.sum(x * x, axis=0, keepdims=True) * (1.0 / N) - m * m
    sc = g1_ref[...] * lax.rsqrt(v + 1e-5)
    sh = be1_ref[...] - m * sc
    xh = jnp.maximum(x * sc + sh, 0.0)
    y = jnp.dot(xh, u2_ref[...], preferred_element_type=F32) + b2_ref[...]
    m2 = jnp.sum(y, axis=0, keepdims=True) * (1.0 / N)
    v2 = jnp.sum(y * y, axis=0, keepdims=True) * (1.0 / N) - m2 * m2
    sc2 = g2_ref[...] * lax.rsqrt(v2 + 1e-5)
    sh2 = be2_ref[...] - m2 * sc2
    out_ref[...] = h + jnp.maximum(y * sc2 + sh2, 0.0)


def _update(h, aggr2, u1t, u1b, b1, g1, be1, u2, b2, g2, be2):
    return pl.pallas_call(
        _update_body,
        out_shape=jax.ShapeDtypeStruct((N, EMB), F32),
    )(h, aggr2, u1t, u1b, b1, g1, be1, u2, b2, g2, be2)


def _pred_body(h_ref, w_ref, b_ref, out_ref):
    hm = jnp.sum(h_ref[...], axis=0, keepdims=True) * (1.0 / N)
    out_ref[...] = jnp.sum(hm * w_ref[...]).reshape(1, 1) + b_ref[...]


def _pred(h, w_row, b11):
    return pl.pallas_call(
        _pred_body,
        out_shape=jax.ShapeDtypeStruct((1, 1), F32),
    )(h, w_row, b11)


def _sc_mesh():
    return plsc.VectorSubcoreMesh(core_axis_name="c", subcore_axis_name="s")


_NW = 32
_CH = 128
_CPW = E // _CH // _NW


_EPW = E // _NW
_RND = 8
_RROWS = _EPW // _RND
_STR = _RROWS // _CH


def _sc_gather(a_tab, b_tab, tf2, i02):
    @functools.partial(
        pl.kernel,
        mesh=_sc_mesh(),
        out_type=[
            jax.ShapeDtypeStruct((E, EMB), F32),
            jax.ShapeDtypeStruct((N, EMB), F32),
        ],
        scratch_types=[
            pltpu.VMEM((_EPW,), I32),
            pltpu.VMEM((_CH,), I32),
            pltpu.VMEM((_RROWS, EMB), F32),
            pltpu.VMEM((_RROWS, EMB), F32),
            pltpu.VMEM((_CH, EMB), F32),
            pltpu.SemaphoreType.DMA,
            pltpu.SemaphoreType.DMA,
            pltpu.SemaphoreType.DMA,
            pltpu.SemaphoreType.DMA,
        ],
    )
    def k(a_hbm, b_hbm, tf_hbm, i0_hbm, xg_hbm, bs_hbm,
          idx_all, idx0_v, rows0, rows1, brows, gsem, wsem0, wsem1, bsem):
        wid = lax.axis_index("s") * 2 + lax.axis_index("c")
        ebase = pl.multiple_of(wid * _EPW, _CH)
        pltpu.sync_copy(tf_hbm.at[wid], idx_all)
        pltpu.sync_copy(i0_hbm.at[wid], idx0_v)
        bcopy = pltpu.async_copy(b_hbm.at[idx0_v], brows, bsem)

        bufs = (rows0, rows1)
        wsems = (wsem0, wsem1)
        wcopies = [None, None]
        for s in range(_RND):
            buf = bufs[s % 2]
            if wcopies[s % 2] is not None:
                wcopies[s % 2].wait()
            gcopies = []
            for b in range(_STR):
                gcopies.append(pltpu.async_copy(
                    a_hbm.at[idx_all.at[pl.ds(s * _RROWS + b * _CH, _CH)]],
                    buf.at[pl.ds(b * _CH, _CH)], gsem))
            for g in gcopies:
                g.wait()
            wcopies[s % 2] = pltpu.async_copy(
                buf, xg_hbm.at[pl.ds(ebase + s * _RROWS, _RROWS)],
                wsems[s % 2])
        bcopy.wait()
        boff = pl.multiple_of(wid * _CH, _CH)
        pltpu.sync_copy(brows, bs_hbm.at[pl.ds(boff, _CH)])
        wcopies[0].wait()
        wcopies[1].wait()

    return k(a_tab, b_tab, tf2, i02)


def _sc_scatter(msg, dst2, zeros):
    @functools.partial(
        pl.kernel,
        mesh=_sc_mesh(),
        out_type=jax.ShapeDtypeStruct((2, N, EMB), F32),
        scratch_types=[
            pltpu.VMEM((_EPW // _CH, _CH), I32),
            pltpu.VMEM((_RROWS, EMB), F32),
            pltpu.VMEM((_RROWS, EMB), F32),
            pltpu.VMEM_SHARED((N, EMB), F32),
            pltpu.SemaphoreType.DMA,
            pltpu.SemaphoreType.DMA,
            pltpu.SemaphoreType.DMA,
        ],
    )
    def k(msg_hbm, dst_hbm, z_hbm, out_hbm, idx_all, rows0, rows1,
          shared, lsem0, lsem1, ssem):
        cidx = lax.axis_index("c")
        sid = lax.axis_index("s")
        wid = sid * 2 + cidx
        ebase = pl.multiple_of(wid * _EPW, _CH)
        zoff = pl.multiple_of(sid * (N // 16), N // 16)
        pltpu.sync_copy(z_hbm.at[pl.ds(zoff, N // 16)],
                        shared.at[pl.ds(zoff, N // 16)])
        pltpu.sync_copy(dst_hbm.at[wid], idx_all)

        bufs = (rows0, rows1)
        lsems = (lsem0, lsem1)
        lcopies = [None, None]
        lcopies[0] = pltpu.async_copy(
            msg_hbm.at[pl.ds(ebase, _RROWS)], rows0, lsem0)
        plsc.subcore_barrier()
        for s in range(_RND):
            if s + 1 < _RND:
                lcopies[(s + 1) % 2] = pltpu.async_copy(
                    msg_hbm.at[pl.ds(ebase + (s + 1) * _RROWS, _RROWS)],
                    bufs[(s + 1) % 2], lsems[(s + 1) % 2])
            lcopies[s % 2].wait()
            buf = bufs[s % 2]
            scopies = []
            for b in range(_STR):
                scopies.append(pltpu.async_copy(
                    buf.at[pl.ds(b * _CH, _CH)],
                    shared.at[idx_all.at[s * _STR + b]], ssem, add=True))
            for sc in scopies:
                sc.wait()

        plsc.subcore_barrier()
        pltpu.sync_copy(shared.at[pl.ds(zoff, N // 16)],
                        out_hbm.at[cidx, pl.ds(zoff, N // 16)])

    return k(msg, dst2, zeros)


def kernel(pos, params):
    pos_p = jnp.pad(pos, ((0, 0), (0, 16 - pos.shape[1])))
    w_in = jnp.pad(params["lin_in_W"], ((0, 16 - pos.shape[1]), (0, 0)))
    h = _lin_in(pos_p, w_in, params["lin_in_b"].reshape(1, EMB))

    zeros = jnp.zeros((N, EMB), F32)
    for p in params["layers"]:
        w1 = p["msg_W1"]
        idx, a_tab, b_tab = _knn(
            h, h.T, w1[:EMB], w1[EMB:], p["msg_b1"].reshape(1, EMB)
        )
        idx = idx[:, :K1]
        to_flat = idx[:, 1:].T.reshape(_NW, _EPW)
        dst3 = to_flat.reshape(_NW, _EPW // _CH, _CH)
        idx0 = idx[:, 0].reshape(_NW, _CH)
        xg, bs = _sc_gather(a_tab, b_tab, to_flat, idx0)
        st1 = _stats1(xg, bs)
        y, st2 = _p2(
            xg, bs, st1,
            p["msg_g1"].reshape(1, EMB), p["msg_be1"].reshape(1, EMB),
            p["msg_W2"], p["msg_b2"].reshape(1, EMB),
        )
        msg = _p3(y, st2, p["msg_g2"].reshape(1, EMB),
                  p["msg_be2"].reshape(1, EMB))
        aggr2 = _sc_scatter(msg, dst3, zeros)
        u1 = p["upd_W1"]
        h = _update(
            h, aggr2, u1[:EMB], u1[EMB:],
            p["upd_b1"].reshape(1, EMB),
            p["upd_g1"].reshape(1, EMB), p["upd_be1"].reshape(1, EMB),
            p["upd_W2"], p["upd_b2"].reshape(1, EMB),
            p["upd_g2"].reshape(1, EMB), p["upd_be2"].reshape(1, EMB),
        )

    out = _pred(h, params["lin_pred_W"].reshape(1, EMB),
                params["lin_pred_b"].reshape(1, 1))
    return out.reshape(-1)


# speedup vs baseline: 1.7885x; 1.7885x over previous
import functools

import jax
import jax.numpy as jnp
from jax import lax
from jax.experimental import pallas as pl
from jax.experimental.pallas import tpu as pltpu
from jax.experimental.pallas import tpu_sc as plsc

N = 4096
EMB = 128
K1 = 17
KN = K1 - 1
E = KN * N
ROWS = 512
NBLK = N // ROWS
EBLK = 16
F32 = jnp.float32
I32 = jnp.int32


def _lin_in_body(pos_ref, w_ref, b_ref, out_ref):
    out_ref[...] = (
        jnp.dot(pos_ref[...], w_ref[...], preferred_element_type=F32)
        + b_ref[...]
    )


def _lin_in(pos_p, w_p, b):
    return pl.pallas_call(
        _lin_in_body,
        out_shape=jax.ShapeDtypeStruct((N, EMB), F32),
    )(pos_p, w_p, b)


def _knn_body(h_ref, ht_ref, w1t_ref, w1b_ref, b1_ref, idx_ref, a_ref, b_ref):
    hb = h_ref[...]
    a_ref[...] = jnp.dot(hb, w1t_ref[...], preferred_element_type=F32)
    b_ref[...] = (
        jnp.dot(hb, w1b_ref[...], preferred_element_type=F32) + b1_ref[...]
    )
    ht = ht_ref[...]
    sq = jnp.sum(ht * ht, axis=0, keepdims=True)
    d = sq - 2.0 * jnp.dot(hb, ht, preferred_element_type=F32)
    iota = lax.broadcasted_iota(I32, (ROWS, N), 1)
    lane = lax.broadcasted_iota(I32, (ROWS, 128), 1)
    bits = lax.bitcast_convert_type(d, I32)
    srt = jnp.where(bits < 0, bits ^ 0x7FFFFFFF, bits)
    key = (srt & ~7) | (iota >> 9)
    BIGI = jnp.int32(0x7FFFFFFF)
    r1 = jnp.full((ROWS, N // 8), BIGI, I32)
    r2 = r1
    r3 = r1
    for t in range(8):
        x = key[:, t * (N // 8):(t + 1) * (N // 8)]
        hi1 = jnp.maximum(r1, x)
        r1 = jnp.minimum(r1, x)
        hi2 = jnp.maximum(r2, hi1)
        r2 = jnp.minimum(r2, hi1)
        r3 = jnp.minimum(r3, hi2)
    iota5 = lax.broadcasted_iota(I32, (ROWS, N // 8), 1)
    acc = jnp.zeros((ROWS, 128), I32)
    flag = jnp.zeros((), jnp.bool_)
    for j in range(K1):
        m = jnp.min(r1, axis=1, keepdims=True)
        flag = jnp.logical_or(flag, jnp.any(m == BIGI))
        candr = jnp.where(r1 == m, iota5, 1 << 30)
        rpos = jnp.min(candr, axis=1, keepdims=True)
        jglob = ((m & 7) << 9) | rpos
        acc = jnp.where(lane == j, jglob, acc)
        atpos = iota5 == rpos
        r1 = jnp.where(atpos, r2, r1)
        r2 = jnp.where(atpos, r3, r2)
        r3 = jnp.where(atpos, BIGI, r3)
    idx_ref[...] = acc

    @pl.when(flag)
    def _():
        dd = d
        accn = jnp.zeros((ROWS, 128), I32)
        for j in range(K1):
            mm = jnp.min(dd, axis=1, keepdims=True)
            cand = jnp.where(dd <= mm, iota, 1 << 30)
            a = jnp.min(cand, axis=1, keepdims=True)
            accn = jnp.where(lane == j, a, accn)
            dd = jnp.where(iota == a, 1.0e30, dd)
        idx_ref[...] = accn


def _knn(h, ht, w1t, w1b, b1):
    return pl.pallas_call(
        _knn_body,
        grid=(NBLK,),
        compiler_params=pltpu.CompilerParams(
            dimension_semantics=("parallel",)),
        in_specs=[
            pl.BlockSpec((ROWS, EMB), lambda i: (i, 0)),
            pl.BlockSpec((EMB, N), lambda i: (0, 0)),
            pl.BlockSpec((EMB, EMB), lambda i: (0, 0)),
            pl.BlockSpec((EMB, EMB), lambda i: (0, 0)),
            pl.BlockSpec((1, EMB), lambda i: (0, 0)),
        ],
        out_specs=[
            pl.BlockSpec((ROWS, 128), lambda i: (i, 0)),
            pl.BlockSpec((ROWS, EMB), lambda i: (i, 0)),
            pl.BlockSpec((ROWS, EMB), lambda i: (i, 0)),
        ],
        out_shape=[
            jax.ShapeDtypeStruct((N, 128), I32),
            jax.ShapeDtypeStruct((N, EMB), F32),
            jax.ShapeDtypeStruct((N, EMB), F32),
        ],
    )(h, ht, w1t, w1b, b1)


def _stats1_body(xg_ref, bs_ref, st_ref):
    i = pl.program_id(0)

    @pl.when(i == 0)
    def _():
        st_ref[...] = jnp.zeros((8, EMB), F32)

    x = xg_ref[...] + bs_ref[...]
    st_ref[0:1, :] += jnp.sum(x, axis=0, keepdims=True)
    st_ref[1:2, :] += jnp.sum(x * x, axis=0, keepdims=True)


def _stats1(xg, bs):
    return pl.pallas_call(
        _stats1_body,
        grid=(EBLK,),
        in_specs=[
            pl.BlockSpec((N, EMB), lambda i: (i, 0)),
            pl.BlockSpec((N, EMB), lambda i: (0, 0)),
        ],
        out_specs=pl.BlockSpec((8, EMB), lambda i: (0, 0)),
        out_shape=jax.ShapeDtypeStruct((8, EMB), F32),
    )(xg, bs)


def _bn_coeffs(st_ref, g_ref, be_ref, count):
    m = st_ref[0:1, :] * (1.0 / count)
    v = st_ref[1:2, :] * (1.0 / count) - m * m
    sc = g_ref[...] * lax.rsqrt(v + 1e-5)
    sh = be_ref[...] - m * sc
    return sc, sh


def _p2_body(xg_ref, bs_ref, st_ref, g_ref, be_ref, w2_ref, b2_ref,
             y_ref, st2_ref):
    i = pl.program_id(0)

    @pl.when(i == 0)
    def _():
        st2_ref[...] = jnp.zeros((8, EMB), F32)

    sc, sh = _bn_coeffs(st_ref, g_ref, be_ref, float(E))
    x = xg_ref[...] + bs_ref[...]
    xh = jnp.maximum(x * sc + sh, 0.0)
    y = jnp.dot(xh, w2_ref[...], preferred_element_type=F32) + b2_ref[...]
    y_ref[...] = y
    st2_ref[0:1, :] += jnp.sum(y, axis=0, keepdims=True)
    st2_ref[1:2, :] += jnp.sum(y * y, axis=0, keepdims=True)


def _p2(xg, bs, st1, g1, be1, w2, b2):
    return pl.pallas_call(
        _p2_body,
        grid=(EBLK,),
        in_specs=[
            pl.BlockSpec((N, EMB), lambda i: (i, 0)),
            pl.BlockSpec((N, EMB), lambda i: (0, 0)),
            pl.BlockSpec((8, EMB), lambda i: (0, 0)),
            pl.BlockSpec((1, EMB), lambda i: (0, 0)),
            pl.BlockSpec((1, EMB), lambda i: (0, 0)),
            pl.BlockSpec((EMB, EMB), lambda i: (0, 0)),
            pl.BlockSpec((1, EMB), lambda i: (0, 0)),
        ],
        out_specs=[
            pl.BlockSpec((N, EMB), lambda i: (i, 0)),
            pl.BlockSpec((8, EMB), lambda i: (0, 0)),
        ],
        out_shape=[
            jax.ShapeDtypeStruct((E, EMB), F32),
            jax.ShapeDtypeStruct((8, EMB), F32),
        ],
    )(xg, bs, st1, g1, be1, w2, b2)


def _p3_body(y_ref, st_ref, g_ref, be_ref, msg_ref):
    sc, sh = _bn_coeffs(st_ref, g_ref, be_ref, float(E))
    msg_ref[...] = jnp.maximum(y_ref[...] * sc + sh, 0.0)


def _p3(y, st2, g2, be2):
    return pl.pallas_call(
        _p3_body,
        grid=(EBLK,),
        compiler_params=pltpu.CompilerParams(
            dimension_semantics=("parallel",)),
        in_specs=[
            pl.BlockSpec((N, EMB), lambda i: (i, 0)),
            pl.BlockSpec((8, EMB), lambda i: (0, 0)),
            pl.BlockSpec((1, EMB), lambda i: (0, 0)),
            pl.BlockSpec((1, EMB), lambda i: (0, 0)),
        ],
        out_specs=pl.BlockSpec((N, EMB), lambda i: (i, 0)),
        out_shape=jax.ShapeDtypeStruct((E, EMB), F32),
    )(y, st2, g2, be2)


def _update_body(h_ref, ag_ref, u1t_ref, u1b_ref, b1_ref, g1_ref, be1_ref,
                 u2_ref, b2_ref, g2_ref, be2_ref, out_ref):
    h = h_ref[...]
    aggr = ag_ref[0] + ag_ref[1]
    x = (
        jnp.dot(h, u1t_ref[...], preferred_element_type=F32)
        + jnp.dot(aggr, u1b_ref[...], preferred_element_type=F32)
        + b1_ref[...]
    )
    m = jnp.sum(x, axis=0, keepdims=True) * (1.0 / N)
    v = jnp.sum(x * x, axis=0, keepdims=True) * (1.0 / N) - m * m
    sc = g1_ref[...] * lax.rsqrt(v + 1e-5)
    sh = be1_ref[...] - m * sc
    xh = jnp.maximum(x * sc + sh, 0.0)
    y = jnp.dot(xh, u2_ref[...], preferred_element_type=F32) + b2_ref[...]
    m2 = jnp.sum(y, axis=0, keepdims=True) * (1.0 / N)
    v2 = jnp.sum(y * y, axis=0, keepdims=True) * (1.0 / N) - m2 * m2
    sc2 = g2_ref[...] * lax.rsqrt(v2 + 1e-5)
    sh2 = be2_ref[...] - m2 * sc2
    out_ref[...] = h + jnp.maximum(y * sc2 + sh2, 0.0)


def _update(h, aggr2, u1t, u1b, b1, g1, be1, u2, b2, g2, be2):
    return pl.pallas_call(
        _update_body,
        out_shape=jax.ShapeDtypeStruct((N, EMB), F32),
    )(h, aggr2, u1t, u1b, b1, g1, be1, u2, b2, g2, be2)


def _pred_body(h_ref, w_ref, b_ref, out_ref):
    hm = jnp.sum(h_ref[...], axis=0, keepdims=True) * (1.0 / N)
    out_ref[...] = jnp.sum(hm * w_ref[...]).reshape(1, 1) + b_ref[...]


def _pred(h, w_row, b11):
    return pl.pallas_call(
        _pred_body,
        out_shape=jax.ShapeDtypeStruct((1, 1), F32),
    )(h, w_row, b11)


def _sc_mesh():
    return plsc.VectorSubcoreMesh(core_axis_name="c", subcore_axis_name="s")


_NW = 32
_CH = 128
_CPW = E // _CH // _NW


_EPW = E // _NW
_RND = 8
_RROWS = _EPW // _RND
_STR = _RROWS // _CH


def _sc_gather(a_tab, b_tab, tf2, i02):
    @functools.partial(
        pl.kernel,
        mesh=_sc_mesh(),
        out_type=[
            jax.ShapeDtypeStruct((E, EMB), F32),
            jax.ShapeDtypeStruct((N, EMB), F32),
        ],
        scratch_types=[
            pltpu.VMEM((_EPW,), I32),
            pltpu.VMEM((_CH,), I32),
            pltpu.VMEM((_RROWS, EMB), F32),
            pltpu.VMEM((_RROWS, EMB), F32),
            pltpu.VMEM((_CH, EMB), F32),
            pltpu.SemaphoreType.DMA,
            pltpu.SemaphoreType.DMA,
            pltpu.SemaphoreType.DMA,
            pltpu.SemaphoreType.DMA,
        ],
    )
    def k(a_hbm, b_hbm, tf_hbm, i0_hbm, xg_hbm, bs_hbm,
          idx_all, idx0_v, rows0, rows1, brows, gsem, wsem0, wsem1, bsem):
        wid = lax.axis_index("s") * 2 + lax.axis_index("c")
        ebase = pl.multiple_of(wid * _EPW, _CH)
        pltpu.sync_copy(tf_hbm.at[wid], idx_all)
        pltpu.sync_copy(i0_hbm.at[wid], idx0_v)
        bcopy = pltpu.async_copy(b_hbm.at[idx0_v], brows, bsem)

        bufs = (rows0, rows1)
        wsems = (wsem0, wsem1)
        wcopies = [None, None]
        for s in range(_RND):
            buf = bufs[s % 2]
            if wcopies[s % 2] is not None:
                wcopies[s % 2].wait()
            gcopies = []
            for b in range(_STR):
                gcopies.append(pltpu.async_copy(
                    a_hbm.at[idx_all.at[pl.ds(s * _RROWS + b * _CH, _CH)]],
                    buf.at[pl.ds(b * _CH, _CH)], gsem))
            for g in gcopies:
                g.wait()
            wcopies[s % 2] = pltpu.async_copy(
                buf, xg_hbm.at[pl.ds(ebase + s * _RROWS, _RROWS)],
                wsems[s % 2])
        bcopy.wait()
        boff = pl.multiple_of(wid * _CH, _CH)
        pltpu.sync_copy(brows, bs_hbm.at[pl.ds(boff, _CH)])
        wcopies[0].wait()
        wcopies[1].wait()

    return k(a_tab, b_tab, tf2, i02)


def _sc_scatter(msg, dst2, zeros):
    @functools.partial(
        pl.kernel,
        mesh=_sc_mesh(),
        out_type=jax.ShapeDtypeStruct((2, N, EMB), F32),
        scratch_types=[
            pltpu.VMEM((_EPW // _CH, _CH), I32),
            pltpu.VMEM((_RROWS, EMB), F32),
            pltpu.VMEM((_RROWS, EMB), F32),
            pltpu.VMEM_SHARED((N, EMB), F32),
            pltpu.SemaphoreType.DMA,
            pltpu.SemaphoreType.DMA,
            pltpu.SemaphoreType.DMA,
        ],
    )
    def k(msg_hbm, dst_hbm, z_hbm, out_hbm, idx_all, rows0, rows1,
          shared, lsem0, lsem1, ssem):
        cidx = lax.axis_index("c")
        sid = lax.axis_index("s")
        wid = sid * 2 + cidx
        ebase = pl.multiple_of(wid * _EPW, _CH)
        zoff = pl.multiple_of(sid * (N // 16), N // 16)
        pltpu.sync_copy(z_hbm.at[pl.ds(zoff, N // 16)],
                        shared.at[pl.ds(zoff, N // 16)])
        pltpu.sync_copy(dst_hbm.at[wid], idx_all)

        bufs = (rows0, rows1)
        lsems = (lsem0, lsem1)
        lcopies = [None, None]
        lcopies[0] = pltpu.async_copy(
            msg_hbm.at[pl.ds(ebase, _RROWS)], rows0, lsem0)
        plsc.subcore_barrier()
        for s in range(_RND):
            if s + 1 < _RND:
                lcopies[(s + 1) % 2] = pltpu.async_copy(
                    msg_hbm.at[pl.ds(ebase + (s + 1) * _RROWS, _RROWS)],
                    bufs[(s + 1) % 2], lsems[(s + 1) % 2])
            lcopies[s % 2].wait()
            buf = bufs[s % 2]
            scopies = []
            for b in range(_STR):
                scopies.append(pltpu.async_copy(
                    buf.at[pl.ds(b * _CH, _CH)],
                    shared.at[idx_all.at[s * _STR + b]], ssem, add=True))
            for sc in scopies:
                sc.wait()

        plsc.subcore_barrier()
        pltpu.sync_copy(shared.at[pl.ds(zoff, N // 16)],
                        out_hbm.at[cidx, pl.ds(zoff, N // 16)])

    return k(msg, dst2, zeros)


def kernel(pos, params):
    pos_p = jnp.pad(pos, ((0, 0), (0, 16 - pos.shape[1])))
    w_in = jnp.pad(params["lin_in_W"], ((0, 16 - pos.shape[1]), (0, 0)))
    h = _lin_in(pos_p, w_in, params["lin_in_b"].reshape(1, EMB))

    zeros = jnp.zeros((N, EMB), F32)
    for p in params["layers"]:
        w1 = p["msg_W1"]
        idx, a_tab, b_tab = _knn(
            h, h.T, w1[:EMB], w1[EMB:], p["msg_b1"].reshape(1, EMB)
        )
        idx = idx[:, :K1]
        to_flat = idx[:, 1:].T.reshape(_NW, _EPW)
        dst3 = to_flat.reshape(_NW, _EPW // _CH, _CH)
        idx0 = idx[:, 0].reshape(_NW, _CH)
        xg, bs = _sc_gather(a_tab, b_tab, to_flat, idx0)
        st1 = _stats1(xg, bs)
        y, st2 = _p2(
            xg, bs, st1,
            p["msg_g1"].reshape(1, EMB), p["msg_be1"].reshape(1, EMB),
            p["msg_W2"], p["msg_b2"].reshape(1, EMB),
        )
        msg = _p3(y, st2, p["msg_g2"].reshape(1, EMB),
                  p["msg_be2"].reshape(1, EMB))
        aggr2 = _sc_scatter(msg, dst3, zeros)
        u1 = p["upd_W1"]
        h = _update(
            h, aggr2, u1[:EMB], u1[EMB:],
            p["upd_b1"].reshape(1, EMB),
            p["upd_g1"].reshape(1, EMB), p["upd_be1"].reshape(1, EMB),
            p["upd_W2"], p["upd_b2"].reshape(1, EMB),
            p["upd_g2"].reshape(1, EMB), p["upd_be2"].reshape(1, EMB),
        )

    out = _pred(h, params["lin_pred_W"].reshape(1, EMB),
                params["lin_pred_b"].reshape(1, 1))
    return out.reshape(-1)


# speedup vs baseline: 1.9345x; 1.0816x over previous
import functools

import jax
import jax.numpy as jnp
from jax import lax
from jax.experimental import pallas as pl
from jax.experimental.pallas import tpu as pltpu
from jax.experimental.pallas import tpu_sc as plsc

N = 4096
EMB = 128
K1 = 17
KN = K1 - 1
E = KN * N
ROWS = 512
NBLK = N // ROWS
EBLK = 16
F32 = jnp.float32
I32 = jnp.int32


def _lin_in_body(pos_ref, w_ref, b_ref, out_ref):
    out_ref[...] = (
        jnp.dot(pos_ref[...], w_ref[...], preferred_element_type=F32)
        + b_ref[...]
    )


def _lin_in(pos_p, w_p, b):
    return pl.pallas_call(
        _lin_in_body,
        out_shape=jax.ShapeDtypeStruct((N, EMB), F32),
    )(pos_p, w_p, b)


def _knn_body(h_ref, ht_ref, w1t_ref, w1b_ref, b1_ref, idx_ref, a_ref, b_ref):
    hb = h_ref[...]
    a_ref[...] = jnp.dot(hb, w1t_ref[...], preferred_element_type=F32)
    b_ref[...] = (
        jnp.dot(hb, w1b_ref[...], preferred_element_type=F32) + b1_ref[...]
    )
    ht = ht_ref[...]
    sq = jnp.sum(ht * ht, axis=0, keepdims=True)
    d = sq - 2.0 * jnp.dot(hb, ht, preferred_element_type=F32)
    iota = lax.broadcasted_iota(I32, (ROWS, N), 1)
    lane = lax.broadcasted_iota(I32, (ROWS, 128), 1)
    bits = lax.bitcast_convert_type(d, I32)
    srt = jnp.where(bits < 0, bits ^ 0x7FFFFFFF, bits)
    key = (srt & ~7) | (iota >> 9)
    BIGI = jnp.int32(0x7FFFFFFF)
    r1 = jnp.full((ROWS, N // 8), BIGI, I32)
    r2 = r1
    r3 = r1
    for t in range(8):
        x = key[:, t * (N // 8):(t + 1) * (N // 8)]
        hi1 = jnp.maximum(r1, x)
        r1 = jnp.minimum(r1, x)
        hi2 = jnp.maximum(r2, hi1)
        r2 = jnp.minimum(r2, hi1)
        r3 = jnp.minimum(r3, hi2)
    iota5 = lax.broadcasted_iota(I32, (ROWS, N // 8), 1)
    acc = jnp.zeros((ROWS, 128), I32)
    flag = jnp.zeros((), jnp.bool_)
    for j in range(K1):
        m = jnp.min(r1, axis=1, keepdims=True)
        flag = jnp.logical_or(flag, jnp.any(m == BIGI))
        candr = jnp.where(r1 == m, iota5, 1 << 30)
        rpos = jnp.min(candr, axis=1, keepdims=True)
        jglob = ((m & 7) << 9) | rpos
        acc = jnp.where(lane == j, jglob, acc)
        atpos = iota5 == rpos
        r1 = jnp.where(atpos, r2, r1)
        r2 = jnp.where(atpos, r3, r2)
        r3 = jnp.where(atpos, BIGI, r3)
    idx_ref[...] = acc

    @pl.when(flag)
    def _():
        dd = d
        accn = jnp.zeros((ROWS, 128), I32)
        for j in range(K1):
            mm = jnp.min(dd, axis=1, keepdims=True)
            cand = jnp.where(dd <= mm, iota, 1 << 30)
            a = jnp.min(cand, axis=1, keepdims=True)
            accn = jnp.where(lane == j, a, accn)
            dd = jnp.where(iota == a, 1.0e30, dd)
        idx_ref[...] = accn


def _knn(h, ht, w1t, w1b, b1):
    return pl.pallas_call(
        _knn_body,
        grid=(NBLK,),
        compiler_params=pltpu.CompilerParams(
            dimension_semantics=("parallel",)),
        in_specs=[
            pl.BlockSpec((ROWS, EMB), lambda i: (i, 0)),
            pl.BlockSpec((EMB, N), lambda i: (0, 0)),
            pl.BlockSpec((EMB, EMB), lambda i: (0, 0)),
            pl.BlockSpec((EMB, EMB), lambda i: (0, 0)),
            pl.BlockSpec((1, EMB), lambda i: (0, 0)),
        ],
        out_specs=[
            pl.BlockSpec((ROWS, 128), lambda i: (i, 0)),
            pl.BlockSpec((ROWS, EMB), lambda i: (i, 0)),
            pl.BlockSpec((ROWS, EMB), lambda i: (i, 0)),
        ],
        out_shape=[
            jax.ShapeDtypeStruct((N, 128), I32),
            jax.ShapeDtypeStruct((N, EMB), F32),
            jax.ShapeDtypeStruct((N, EMB), F32),
        ],
    )(h, ht, w1t, w1b, b1)


def _bn_coeffs(st_ref, g_ref, be_ref, count):
    m = st_ref[0:1, :] * (1.0 / count)
    v = st_ref[1:2, :] * (1.0 / count) - m * m
    sc = g_ref[...] * lax.rsqrt(v + 1e-5)
    sh = be_ref[...] - m * sc
    return sc, sh


def _edge_body(xg_ref, bs_ref, g1_ref, be1_ref, w2_ref, b2_ref,
               g2_ref, be2_ref, msg_ref, st1_ref, y_ref, st2_ref):
    p = pl.program_id(0)
    i = pl.program_id(1)

    @pl.when((p == 0) & (i == 0))
    def _():
        st1_ref[...] = jnp.zeros((8, EMB), F32)
        st2_ref[...] = jnp.zeros((8, EMB), F32)

    @pl.when(p == 0)
    def _():
        x = xg_ref[...] + bs_ref[...]
        st1_ref[0:1, :] += jnp.sum(x, axis=0, keepdims=True)
        st1_ref[1:2, :] += jnp.sum(x * x, axis=0, keepdims=True)

    @pl.when(p == 1)
    def _():
        sc, sh = _bn_coeffs(st1_ref, g1_ref, be1_ref, float(E))
        x = xg_ref[...] + bs_ref[...]
        xh = jnp.maximum(x * sc + sh, 0.0)
        y = jnp.dot(xh, w2_ref[...], preferred_element_type=F32) + b2_ref[...]
        y_ref[pl.ds(i * N, N), :] = y
        st2_ref[0:1, :] += jnp.sum(y, axis=0, keepdims=True)
        st2_ref[1:2, :] += jnp.sum(y * y, axis=0, keepdims=True)

    @pl.when(p == 2)
    def _():
        sc2, sh2 = _bn_coeffs(st2_ref, g2_ref, be2_ref, float(E))
        msg_ref[...] = jnp.maximum(
            y_ref[pl.ds(i * N, N), :] * sc2 + sh2, 0.0)


def _edge_mlp(xg, bs, g1, be1, w2, b2, g2, be2):
    xmap = lambda p, i: (jnp.where(p == 2, 0, i), 0)
    cmap = lambda p, i: (0, 0)
    return pl.pallas_call(
        _edge_body,
        grid=(3, EBLK),
        in_specs=[
            pl.BlockSpec((N, EMB), xmap),
            pl.BlockSpec((N, EMB), cmap),
            pl.BlockSpec((1, EMB), cmap),
            pl.BlockSpec((1, EMB), cmap),
            pl.BlockSpec((EMB, EMB), cmap),
            pl.BlockSpec((1, EMB), cmap),
            pl.BlockSpec((1, EMB), cmap),
            pl.BlockSpec((1, EMB), cmap),
        ],
        out_specs=pl.BlockSpec((N, EMB), lambda p, i: (jnp.where(p < 2, 0, i), 0)),
        out_shape=jax.ShapeDtypeStruct((E, EMB), F32),
        scratch_shapes=[
            pltpu.VMEM((8, EMB), F32),
            pltpu.VMEM((E, EMB), F32),
            pltpu.VMEM((8, EMB), F32),
        ],
    )(xg, bs, g1, be1, w2, b2, g2, be2)


def _update_body(h_ref, ag_ref, u1t_ref, u1b_ref, b1_ref, g1_ref, be1_ref,
                 u2_ref, b2_ref, g2_ref, be2_ref, wp_ref, bp_ref,
                 out_ref, pr_ref):
    h = h_ref[...]
    aggr = ag_ref[0] + ag_ref[1]
    x = (
        jnp.dot(h, u1t_ref[...], preferred_element_type=F32)
        + jnp.dot(aggr, u1b_ref[...], preferred_element_type=F32)
        + b1_ref[...]
    )
    m = jnp.sum(x, axis=0, keepdims=True) * (1.0 / N)
    v = jnp.sum(x * x, axis=0, keepdims=True) * (1.0 / N) - m * m
    sc = g1_ref[...] * lax.rsqrt(v + 1e-5)
    sh = be1_ref[...] - m * sc
    xh = jnp.maximum(x * sc + sh, 0.0)
    y = jnp.dot(xh, u2_ref[...], preferred_element_type=F32) + b2_ref[...]
    m2 = jnp.sum(y, axis=0, keepdims=True) * (1.0 / N)
    v2 = jnp.sum(y * y, axis=0, keepdims=True) * (1.0 / N) - m2 * m2
    sc2 = g2_ref[...] * lax.rsqrt(v2 + 1e-5)
    sh2 = be2_ref[...] - m2 * sc2
    hn = h + jnp.maximum(y * sc2 + sh2, 0.0)
    out_ref[...] = hn
    hm = jnp.sum(hn, axis=0, keepdims=True) * (1.0 / N)
    pr_ref[...] = jnp.sum(hm * wp_ref[...]).reshape(1, 1) + bp_ref[...]


def _update(h, aggr2, u1t, u1b, b1, g1, be1, u2, b2, g2, be2, wp, bp):
    return pl.pallas_call(
        _update_body,
        out_shape=[
            jax.ShapeDtypeStruct((N, EMB), F32),
            jax.ShapeDtypeStruct((1, 1), F32),
        ],
    )(h, aggr2, u1t, u1b, b1, g1, be1, u2, b2, g2, be2, wp, bp)


def _sc_mesh():
    return plsc.VectorSubcoreMesh(core_axis_name="c", subcore_axis_name="s")


_NW = 32
_CH = 128
_CPW = E // _CH // _NW


_EPW = E // _NW
_RND = 8
_RROWS = _EPW // _RND
_STR = _RROWS // _CH


def _sc_gather(a_tab, b_tab, tf2, i02):
    @functools.partial(
        pl.kernel,
        mesh=_sc_mesh(),
        out_type=[
            jax.ShapeDtypeStruct((E, EMB), F32),
            jax.ShapeDtypeStruct((N, EMB), F32),
        ],
        scratch_types=[
            pltpu.VMEM((_EPW,), I32),
            pltpu.VMEM((_CH,), I32),
            pltpu.VMEM((_RROWS, EMB), F32),
            pltpu.VMEM((_RROWS, EMB), F32),
            pltpu.VMEM((_CH, EMB), F32),
            pltpu.SemaphoreType.DMA,
            pltpu.SemaphoreType.DMA,
            pltpu.SemaphoreType.DMA,
            pltpu.SemaphoreType.DMA,
        ],
    )
    def k(a_hbm, b_hbm, tf_hbm, i0_hbm, xg_hbm, bs_hbm,
          idx_all, idx0_v, rows0, rows1, brows, gsem, wsem0, wsem1, bsem):
        wid = lax.axis_index("s") * 2 + lax.axis_index("c")
        ebase = pl.multiple_of(wid * _EPW, _CH)
        pltpu.sync_copy(tf_hbm.at[wid], idx_all)
        pltpu.sync_copy(i0_hbm.at[wid], idx0_v)
        bcopy = pltpu.async_copy(b_hbm.at[idx0_v], brows, bsem)

        bufs = (rows0, rows1)
        wsems = (wsem0, wsem1)
        wcopies = [None, None]
        for s in range(_RND):
            buf = bufs[s % 2]
            if wcopies[s % 2] is not None:
                wcopies[s % 2].wait()
            gcopies = []
            for b in range(_STR):
                gcopies.append(pltpu.async_copy(
                    a_hbm.at[idx_all.at[pl.ds(s * _RROWS + b * _CH, _CH)]],
                    buf.at[pl.ds(b * _CH, _CH)], gsem))
            for g in gcopies:
                g.wait()
            wcopies[s % 2] = pltpu.async_copy(
                buf, xg_hbm.at[pl.ds(ebase + s * _RROWS, _RROWS)],
                wsems[s % 2])
        bcopy.wait()
        boff = pl.multiple_of(wid * _CH, _CH)
        pltpu.sync_copy(brows, bs_hbm.at[pl.ds(boff, _CH)])
        wcopies[0].wait()
        wcopies[1].wait()

    return k(a_tab, b_tab, tf2, i02)


def _sc_scatter(msg, dst2, zeros):
    @functools.partial(
        pl.kernel,
        mesh=_sc_mesh(),
        out_type=jax.ShapeDtypeStruct((2, N, EMB), F32),
        scratch_types=[
            pltpu.VMEM((_EPW // _CH, _CH), I32),
            pltpu.VMEM((_RROWS, EMB), F32),
            pltpu.VMEM((_RROWS, EMB), F32),
            pltpu.VMEM_SHARED((N, EMB), F32),
            pltpu.SemaphoreType.DMA,
            pltpu.SemaphoreType.DMA,
            pltpu.SemaphoreType.DMA,
        ],
    )
    def k(msg_hbm, dst_hbm, z_hbm, out_hbm, idx_all, rows0, rows1,
          shared, lsem0, lsem1, ssem):
        cidx = lax.axis_index("c")
        sid = lax.axis_index("s")
        wid = sid * 2 + cidx
        ebase = pl.multiple_of(wid * _EPW, _CH)
        zoff = pl.multiple_of(sid * (N // 16), N // 16)
        pltpu.sync_copy(z_hbm.at[pl.ds(zoff, N // 16)],
                        shared.at[pl.ds(zoff, N // 16)])
        pltpu.sync_copy(dst_hbm.at[wid], idx_all)

        bufs = (rows0, rows1)
        lsems = (lsem0, lsem1)
        lcopies = [None, None]
        lcopies[0] = pltpu.async_copy(
            msg_hbm.at[pl.ds(ebase, _RROWS)], rows0, lsem0)
        plsc.subcore_barrier()
        for s in range(_RND):
            if s + 1 < _RND:
                lcopies[(s + 1) % 2] = pltpu.async_copy(
                    msg_hbm.at[pl.ds(ebase + (s + 1) * _RROWS, _RROWS)],
                    bufs[(s + 1) % 2], lsems[(s + 1) % 2])
            lcopies[s % 2].wait()
            buf = bufs[s % 2]
            scopies = []
            for b in range(_STR):
                scopies.append(pltpu.async_copy(
                    buf.at[pl.ds(b * _CH, _CH)],
                    shared.at[idx_all.at[s * _STR + b]], ssem, add=True))
            for sc in scopies:
                sc.wait()

        plsc.subcore_barrier()
        pltpu.sync_copy(shared.at[pl.ds(zoff, N // 16)],
                        out_hbm.at[cidx, pl.ds(zoff, N // 16)])

    return k(msg, dst2, zeros)


def kernel(pos, params):
    pos_p = jnp.pad(pos, ((0, 0), (0, 16 - pos.shape[1])))
    w_in = jnp.pad(params["lin_in_W"], ((0, 16 - pos.shape[1]), (0, 0)))
    h = _lin_in(pos_p, w_in, params["lin_in_b"].reshape(1, EMB))

    zeros = jnp.zeros((N, EMB), F32)
    for p in params["layers"]:
        w1 = p["msg_W1"]
        idx, a_tab, b_tab = _knn(
            h, h.T, w1[:EMB], w1[EMB:], p["msg_b1"].reshape(1, EMB)
        )
        idx = idx[:, :K1]
        to_flat = idx[:, 1:].T.reshape(_NW, _EPW)
        dst3 = to_flat.reshape(_NW, _EPW // _CH, _CH)
        idx0 = idx[:, 0].reshape(_NW, _CH)
        xg, bs = _sc_gather(a_tab, b_tab, to_flat, idx0)
        msg = _edge_mlp(
            xg, bs,
            p["msg_g1"].reshape(1, EMB), p["msg_be1"].reshape(1, EMB),
            p["msg_W2"], p["msg_b2"].reshape(1, EMB),
            p["msg_g2"].reshape(1, EMB), p["msg_be2"].reshape(1, EMB),
        )
        aggr2 = _sc_scatter(msg, dst3, zeros)
        u1 = p["upd_W1"]
        h, out = _update(
            h, aggr2, u1[:EMB], u1[EMB:],
            p["upd_b1"].reshape(1, EMB),
            p["upd_g1"].reshape(1, EMB), p["upd_be1"].reshape(1, EMB),
            p["upd_W2"], p["upd_b2"].reshape(1, EMB),
            p["upd_g2"].reshape(1, EMB), p["upd_be2"].reshape(1, EMB),
            params["lin_pred_W"].reshape(1, EMB),
            params["lin_pred_b"].reshape(1, 1),
        )

    return out.reshape(-1)


# speedup vs baseline: 2.0398x; 1.0544x over previous
import functools

import jax
import jax.numpy as jnp
from jax import lax
from jax.experimental import pallas as pl
from jax.experimental.pallas import tpu as pltpu
from jax.experimental.pallas import tpu_sc as plsc

N = 4096
EMB = 128
K1 = 17
KN = K1 - 1
E = KN * N
ROWS = 512
NBLK = N // ROWS
EBLK = 16
F32 = jnp.float32
I32 = jnp.int32


def _lin_in_body(pos_ref, w_ref, b_ref, out_ref):
    out_ref[...] = (
        jnp.dot(pos_ref[...], w_ref[...], preferred_element_type=F32)
        + b_ref[...]
    )


def _lin_in(pos_p, w_p, b):
    return pl.pallas_call(
        _lin_in_body,
        out_shape=jax.ShapeDtypeStruct((N, EMB), F32),
    )(pos_p, w_p, b)


def _knn_body(h_ref, ht_ref, w1t_ref, w1b_ref, b1_ref, idx_ref, a_ref, b_ref):
    hb = h_ref[...]
    a_ref[...] = jnp.dot(hb, w1t_ref[...], preferred_element_type=F32)
    b_ref[...] = (
        jnp.dot(hb, w1b_ref[...], preferred_element_type=F32) + b1_ref[...]
    )
    ht = ht_ref[...]
    sq = jnp.sum(ht * ht, axis=0, keepdims=True)
    d = sq - 2.0 * jnp.dot(hb, ht, preferred_element_type=F32)
    iota = lax.broadcasted_iota(I32, (ROWS, N), 1)
    lane = lax.broadcasted_iota(I32, (ROWS, 128), 1)
    bits = lax.bitcast_convert_type(d, I32)
    srt = jnp.where(bits < 0, bits ^ 0x7FFFFFFF, bits)
    key = (srt & ~7) | (iota >> 9)
    BIGI = jnp.int32(0x7FFFFFFF)
    r1 = jnp.full((ROWS, N // 8), BIGI, I32)
    r2 = r1
    r3 = r1
    for t in range(8):
        x = key[:, t * (N // 8):(t + 1) * (N // 8)]
        hi1 = jnp.maximum(r1, x)
        r1 = jnp.minimum(r1, x)
        hi2 = jnp.maximum(r2, hi1)
        r2 = jnp.minimum(r2, hi1)
        r3 = jnp.minimum(r3, hi2)
    iota5 = lax.broadcasted_iota(I32, (ROWS, N // 8), 1)
    acc = jnp.zeros((ROWS, 128), I32)
    flag = jnp.zeros((), jnp.bool_)
    for j in range(K1):
        m = jnp.min(r1, axis=1, keepdims=True)
        flag = jnp.logical_or(flag, jnp.any(m == BIGI))
        candr = jnp.where(r1 == m, iota5, 1 << 30)
        rpos = jnp.min(candr, axis=1, keepdims=True)
        jglob = ((m & 7) << 9) | rpos
        acc = jnp.where(lane == j, jglob, acc)
        atpos = iota5 == rpos
        r1 = jnp.where(atpos, r2, r1)
        r2 = jnp.where(atpos, r3, r2)
        r3 = jnp.where(atpos, BIGI, r3)
    idx_ref[...] = acc

    @pl.when(flag)
    def _():
        dd = d
        accn = jnp.zeros((ROWS, 128), I32)
        for j in range(K1):
            mm = jnp.min(dd, axis=1, keepdims=True)
            cand = jnp.where(dd <= mm, iota, 1 << 30)
            a = jnp.min(cand, axis=1, keepdims=True)
            accn = jnp.where(lane == j, a, accn)
            dd = jnp.where(iota == a, 1.0e30, dd)
        idx_ref[...] = accn


def _knn(h, ht, w1t, w1b, b1):
    return pl.pallas_call(
        _knn_body,
        grid=(NBLK,),
        compiler_params=pltpu.CompilerParams(
            dimension_semantics=("parallel",)),
        in_specs=[
            pl.BlockSpec((ROWS, EMB), lambda i: (i, 0)),
            pl.BlockSpec((EMB, N), lambda i: (0, 0)),
            pl.BlockSpec((EMB, EMB), lambda i: (0, 0)),
            pl.BlockSpec((EMB, EMB), lambda i: (0, 0)),
            pl.BlockSpec((1, EMB), lambda i: (0, 0)),
        ],
        out_specs=[
            pl.BlockSpec((ROWS, 128), lambda i: (i, 0)),
            pl.BlockSpec((ROWS, EMB), lambda i: (i, 0)),
            pl.BlockSpec((ROWS, EMB), lambda i: (i, 0)),
        ],
        out_shape=[
            jax.ShapeDtypeStruct((N, 128), I32),
            jax.ShapeDtypeStruct((N, EMB), F32),
            jax.ShapeDtypeStruct((N, EMB), F32),
        ],
    )(h, ht, w1t, w1b, b1)


def _bn_coeffs(st_ref, g_ref, be_ref, count):
    m = st_ref[0:1, :] * (1.0 / count)
    v = st_ref[1:2, :] * (1.0 / count) - m * m
    sc = g_ref[...] * lax.rsqrt(v + 1e-5)
    sh = be_ref[...] - m * sc
    return sc, sh


def _edge_body(xg_ref, bs_ref, g1_ref, be1_ref, w2_ref, b2_ref,
               g2_ref, be2_ref, msg_ref, st1_ref, y_ref, st2_ref):
    p = pl.program_id(0)
    i = pl.program_id(1)

    @pl.when((p == 0) & (i == 0))
    def _():
        st1_ref[...] = jnp.zeros((8, EMB), F32)
        st2_ref[...] = jnp.zeros((8, EMB), F32)

    @pl.when(p == 0)
    def _():
        x = xg_ref[...] + bs_ref[...]
        st1_ref[0:1, :] += jnp.sum(x, axis=0, keepdims=True)
        st1_ref[1:2, :] += jnp.sum(x * x, axis=0, keepdims=True)

    @pl.when(p == 1)
    def _():
        sc, sh = _bn_coeffs(st1_ref, g1_ref, be1_ref, float(E))
        x = xg_ref[...] + bs_ref[...]
        xh = jnp.maximum(x * sc + sh, 0.0)
        y = jnp.dot(xh, w2_ref[...], preferred_element_type=F32) + b2_ref[...]
        y_ref[pl.ds(i * N, N), :] = y
        st2_ref[0:1, :] += jnp.sum(y, axis=0, keepdims=True)
        st2_ref[1:2, :] += jnp.sum(y * y, axis=0, keepdims=True)

    @pl.when(p == 2)
    def _():
        sc2, sh2 = _bn_coeffs(st2_ref, g2_ref, be2_ref, float(E))
        msg_ref[...] = jnp.maximum(
            y_ref[pl.ds(i * N, N), :] * sc2 + sh2, 0.0)


def _edge_mlp(xg, bs, g1, be1, w2, b2, g2, be2):
    xmap = lambda p, i: (jnp.where(p == 2, 0, i), 0)
    cmap = lambda p, i: (0, 0)
    return pl.pallas_call(
        _edge_body,
        grid=(3, EBLK),
        in_specs=[
            pl.BlockSpec((N, EMB), xmap),
            pl.BlockSpec((N, EMB), cmap),
            pl.BlockSpec((1, EMB), cmap),
            pl.BlockSpec((1, EMB), cmap),
            pl.BlockSpec((EMB, EMB), cmap),
            pl.BlockSpec((1, EMB), cmap),
            pl.BlockSpec((1, EMB), cmap),
            pl.BlockSpec((1, EMB), cmap),
        ],
        out_specs=pl.BlockSpec((N, EMB), lambda p, i: (jnp.where(p < 2, 0, i), 0)),
        out_shape=jax.ShapeDtypeStruct((E, EMB), F32),
        scratch_shapes=[
            pltpu.VMEM((8, EMB), F32),
            pltpu.VMEM((E, EMB), F32),
            pltpu.VMEM((8, EMB), F32),
        ],
    )(xg, bs, g1, be1, w2, b2, g2, be2)


def _update_body(h_ref, ag_ref, u1t_ref, u1b_ref, b1_ref, g1_ref, be1_ref,
                 u2_ref, b2_ref, g2_ref, be2_ref, wp_ref, bp_ref,
                 out_ref, pr_ref):
    h = h_ref[...]
    aggr = ag_ref[0] + ag_ref[1]
    x = (
        jnp.dot(h, u1t_ref[...], preferred_element_type=F32)
        + jnp.dot(aggr, u1b_ref[...], preferred_element_type=F32)
        + b1_ref[...]
    )
    m = jnp.sum(x, axis=0, keepdims=True) * (1.0 / N)
    v = jnp.sum(x * x, axis=0, keepdims=True) * (1.0 / N) - m * m
    sc = g1_ref[...] * lax.rsqrt(v + 1e-5)
    sh = be1_ref[...] - m * sc
    xh = jnp.maximum(x * sc + sh, 0.0)
    y = jnp.dot(xh, u2_ref[...], preferred_element_type=F32) + b2_ref[...]
    m2 = jnp.sum(y, axis=0, keepdims=True) * (1.0 / N)
    v2 = jnp.sum(y * y, axis=0, keepdims=True) * (1.0 / N) - m2 * m2
    sc2 = g2_ref[...] * lax.rsqrt(v2 + 1e-5)
    sh2 = be2_ref[...] - m2 * sc2
    hn = h + jnp.maximum(y * sc2 + sh2, 0.0)
    out_ref[...] = hn
    hm = jnp.sum(hn, axis=0, keepdims=True) * (1.0 / N)
    pr_ref[...] = jnp.sum(hm * wp_ref[...]).reshape(1, 1) + bp_ref[...]


def _update(h, aggr2, u1t, u1b, b1, g1, be1, u2, b2, g2, be2, wp, bp):
    return pl.pallas_call(
        _update_body,
        out_shape=[
            jax.ShapeDtypeStruct((N, EMB), F32),
            jax.ShapeDtypeStruct((1, 1), F32),
        ],
    )(h, aggr2, u1t, u1b, b1, g1, be1, u2, b2, g2, be2, wp, bp)


def _sc_mesh():
    return plsc.VectorSubcoreMesh(core_axis_name="c", subcore_axis_name="s")


_NW = 32
_CH = 128
_CPW = E // _CH // _NW


_EPW = E // _NW
_RND = 8
_RROWS = _EPW // _RND
_STR = _RROWS // _CH


def _sc_gather(a_tab, b_tab, tf2, i02):
    nchunks = _EPW // _CH

    @functools.partial(
        pl.kernel,
        mesh=_sc_mesh(),
        out_type=[
            jax.ShapeDtypeStruct((E, EMB), F32),
            jax.ShapeDtypeStruct((N, EMB), F32),
        ],
        scratch_types=[
            pltpu.VMEM((_EPW,), I32),
            pltpu.VMEM((_CH,), I32),
            pltpu.VMEM((_CH, EMB), F32),
            pltpu.VMEM((_CH, EMB), F32),
            pltpu.VMEM((_CH, EMB), F32),
            pltpu.VMEM((_CH, EMB), F32),
            pltpu.VMEM((_CH, EMB), F32),
            pltpu.VMEM_SHARED((N, EMB), F32),
            pltpu.SemaphoreType.DMA((4,)),
            pltpu.SemaphoreType.DMA((4,)),
            pltpu.SemaphoreType.DMA,
        ],
    )
    def k(a_hbm, b_hbm, tf_hbm, i0_hbm, xg_hbm, bs_hbm,
          idx_all, idx0_v, buf0, buf1, buf2, buf3, brows, ash,
          gsems, wsems, bsem):
        cidx = lax.axis_index("c")
        sid = lax.axis_index("s")
        wid = sid * 2 + cidx
        ebase = pl.multiple_of(wid * _EPW, _CH)
        soff = pl.multiple_of(sid * (N // 16), N // 16)
        pltpu.sync_copy(a_hbm.at[pl.ds(soff, N // 16)],
                        ash.at[pl.ds(soff, N // 16)])
        pltpu.sync_copy(tf_hbm.at[wid], idx_all)
        pltpu.sync_copy(i0_hbm.at[wid], idx0_v)
        bcopy = pltpu.async_copy(b_hbm.at[idx0_v], brows, bsem)
        plsc.subcore_barrier()

        bufs = (buf0, buf1, buf2, buf3)
        gc = [None] * nchunks
        wc = [None] * nchunks
        for c in range(nchunks):
            b = c % 4
            if c >= 4:
                wc[c - 4].wait()
            gc[c] = pltpu.async_copy(
                ash.at[idx_all.at[pl.ds(c * _CH, _CH)]],
                bufs[b], gsems.at[b])
            if c >= 2:
                gc[c - 2].wait()
                wc[c - 2] = pltpu.async_copy(
                    bufs[(c - 2) % 4],
                    xg_hbm.at[pl.ds(ebase + (c - 2) * _CH, _CH)],
                    wsems.at[(c - 2) % 4])
        for c in (nchunks - 2, nchunks - 1):
            gc[c].wait()
            wc[c] = pltpu.async_copy(
                bufs[c % 4],
                xg_hbm.at[pl.ds(ebase + c * _CH, _CH)],
                wsems.at[c % 4])
        bcopy.wait()
        boff = pl.multiple_of(wid * _CH, _CH)
        pltpu.sync_copy(brows, bs_hbm.at[pl.ds(boff, _CH)])
        for c in range(nchunks - 4, nchunks):
            wc[c].wait()

    return k(a_tab, b_tab, tf2, i02)


def _sc_scatter(msg, dst2, zeros):
    @functools.partial(
        pl.kernel,
        mesh=_sc_mesh(),
        out_type=jax.ShapeDtypeStruct((2, N, EMB), F32),
        scratch_types=[
            pltpu.VMEM((_EPW // _CH, _CH), I32),
            pltpu.VMEM((_CH, EMB), F32),
            pltpu.VMEM((_CH, EMB), F32),
            pltpu.VMEM((_CH, EMB), F32),
            pltpu.VMEM((_CH, EMB), F32),
            pltpu.VMEM_SHARED((N, EMB), F32),
            pltpu.SemaphoreType.DMA((4,)),
            pltpu.SemaphoreType.DMA((4,)),
        ],
    )
    def k(msg_hbm, dst_hbm, z_hbm, out_hbm, idx_all, buf0, buf1, buf2, buf3,
          shared, lsems, ssems):
        cidx = lax.axis_index("c")
        sid = lax.axis_index("s")
        wid = sid * 2 + cidx
        ebase = pl.multiple_of(wid * _EPW, _CH)
        nchunks = _EPW // _CH
        zoff = pl.multiple_of(sid * (N // 16), N // 16)
        pltpu.sync_copy(z_hbm.at[pl.ds(zoff, N // 16)],
                        shared.at[pl.ds(zoff, N // 16)])
        pltpu.sync_copy(dst_hbm.at[wid], idx_all)

        bufs = (buf0, buf1, buf2, buf3)
        lc = [None] * nchunks
        sc = [None] * nchunks
        lc[0] = pltpu.async_copy(
            msg_hbm.at[pl.ds(ebase, _CH)], buf0, lsems.at[0])
        lc[1] = pltpu.async_copy(
            msg_hbm.at[pl.ds(ebase + _CH, _CH)], buf1, lsems.at[1])
        plsc.subcore_barrier()
        for c in range(2, nchunks):
            b = c % 4
            if c >= 4:
                sc[c - 4].wait()
            lc[c] = pltpu.async_copy(
                msg_hbm.at[pl.ds(ebase + c * _CH, _CH)],
                bufs[b], lsems.at[b])
            lc[c - 2].wait()
            sc[c - 2] = pltpu.async_copy(
                bufs[(c - 2) % 4],
                shared.at[idx_all.at[c - 2]], ssems.at[(c - 2) % 4],
                add=True)
        for c in (nchunks - 2, nchunks - 1):
            lc[c].wait()
            sc[c] = pltpu.async_copy(
                bufs[c % 4], shared.at[idx_all.at[c]], ssems.at[c % 4],
                add=True)
        for c in range(nchunks - 4, nchunks):
            sc[c].wait()

        plsc.subcore_barrier()
        pltpu.sync_copy(shared.at[pl.ds(zoff, N // 16)],
                        out_hbm.at[cidx, pl.ds(zoff, N // 16)])

    return k(msg, dst2, zeros)


def kernel(pos, params):
    pos_p = jnp.pad(pos, ((0, 0), (0, 16 - pos.shape[1])))
    w_in = jnp.pad(params["lin_in_W"], ((0, 16 - pos.shape[1]), (0, 0)))
    h = _lin_in(pos_p, w_in, params["lin_in_b"].reshape(1, EMB))

    zeros = jnp.zeros((N, EMB), F32)
    for p in params["layers"]:
        w1 = p["msg_W1"]
        idx, a_tab, b_tab = _knn(
            h, h.T, w1[:EMB], w1[EMB:], p["msg_b1"].reshape(1, EMB)
        )
        idx = idx[:, :K1]
        to_flat = idx[:, 1:].T.reshape(_NW, _EPW)
        dst3 = to_flat.reshape(_NW, _EPW // _CH, _CH)
        idx0 = idx[:, 0].reshape(_NW, _CH)
        xg, bs = _sc_gather(a_tab, b_tab, to_flat, idx0)
        msg = _edge_mlp(
            xg, bs,
            p["msg_g1"].reshape(1, EMB), p["msg_be1"].reshape(1, EMB),
            p["msg_W2"], p["msg_b2"].reshape(1, EMB),
            p["msg_g2"].reshape(1, EMB), p["msg_be2"].reshape(1, EMB),
        )
        aggr2 = _sc_scatter(msg, dst3, zeros)
        u1 = p["upd_W1"]
        h, out = _update(
            h, aggr2, u1[:EMB], u1[EMB:],
            p["upd_b1"].reshape(1, EMB),
            p["upd_g1"].reshape(1, EMB), p["upd_be1"].reshape(1, EMB),
            p["upd_W2"], p["upd_b2"].reshape(1, EMB),
            p["upd_g2"].reshape(1, EMB), p["upd_be2"].reshape(1, EMB),
            params["lin_pred_W"].reshape(1, EMB),
            params["lin_pred_b"].reshape(1, 1),
        )

    return out.reshape(-1)
